# baseline (device time: 619735 ns/iter reference)
import jax
import jax.numpy as jnp
from jax import lax
from jax.experimental import pallas as pl
from jax.experimental.pallas import tpu as pltpu

M = 4096
N = 4096
BLK = M // 8

AXIS_W = {"x": 4, "y": 2, "z": 1}
SCHEDULES = (("z", "y", "x"), ("y", "x", "z"), ("x", "z", "y"))
COL0 = (0, 384, 1920)
COLW = (384, 1536, 2176)
N_SEMS = 18


def _runs(exchanged):
    offs = [0]
    for a in exchanged:
        w = AXIS_W[a]
        offs = sorted(o + d for o in offs for d in (0, w))
    runs = []
    start, length = offs[0], 1
    for o in offs[1:]:
        if o == start + length:
            length += 1
        else:
            runs.append((start, length))
            start, length = o, 1
    runs.append((start, length))
    return runs


def _allreduce_body(own_ref, send_ref, out_ref, rs_buf, send_sems, recv_sems,
                    copy_sem):
    x = lax.axis_index("x")
    y = lax.axis_index("y")
    z = lax.axis_index("z")
    idx = {"x": x, "y": y, "z": z}
    m = 2 * y + z
    b_own = 4 * x + m

    def nbr_of(a):
        return tuple(1 - idx[ax] if ax == a else idx[ax] for ax in "xyz")

    barrier = pltpu.get_barrier_semaphore()
    for a in "xyz":
        pl.semaphore_signal(barrier, inc=1, device_id=nbr_of(a),
                            device_id_type=pl.DeviceIdType.MESH)
    pl.semaphore_wait(barrier, 3)

    rs_descs = []
    for s in range(3):
        cols = pl.ds(COL0[s], COLW[s])
        d = pltpu.make_async_remote_copy(
            src_ref=send_ref.at[:, cols],
            dst_ref=rs_buf.at[:, cols],
            send_sem=send_sems.at[s],
            recv_sem=recv_sems.at[s],
            device_id=nbr_of("x"),
            device_id_type=pl.DeviceIdType.MESH,
        )
        d.start()
        rs_descs.append(d)

    sem_i = [3]
    exchanged = [(), (), ()]
    pend = [(), (), ()]

    def start_hop(s, h):
        a = SCHEDULES[s][h]
        ex = exchanged[s]
        base = sum(AXIS_W[ax] * idx[ax] for ax in "xyz" if ax not in ex)
        cols = pl.ds(COL0[s], COLW[s])
        descs = []
        for off, length in _runs(ex):
            rows = pl.ds((base + off) * BLK, length * BLK)
            d = pltpu.make_async_remote_copy(
                src_ref=out_ref.at[rows, cols],
                dst_ref=out_ref.at[rows, cols],
                send_sem=send_sems.at[sem_i[0]],
                recv_sem=recv_sems.at[sem_i[0]],
                device_id=nbr_of(a),
                device_id_type=pl.DeviceIdType.MESH,
            )
            d.start()
            descs.append(d)
            sem_i[0] += 1
        exchanged[s] = ex + (a,)
        pend[s] = tuple(descs)

    own_rows = pl.ds(b_own * BLK, BLK)
    for s in range(3):
        cols = pl.ds(COL0[s], COLW[s])
        rs_descs[s].wait()
        rs_buf[:, cols] = own_ref[:, cols] + rs_buf[:, cols]
        cp = pltpu.make_async_copy(
            rs_buf.at[:, cols], out_ref.at[own_rows, cols], copy_sem)
        cp.start()
        cp.wait()
        start_hop(s, 0)

    for h in (1, 2):
        for s in range(3):
            for d in pend[s]:
                d.wait()
            start_hop(s, h)
    for s in range(3):
        for d in pend[s]:
            d.wait()


def _allreduce(own, send):
    return pl.pallas_call(
        _allreduce_body,
        out_shape=jax.ShapeDtypeStruct((M, N), jnp.float32),
        in_specs=[
            pl.BlockSpec(memory_space=pltpu.VMEM),
            pl.BlockSpec(memory_space=pltpu.VMEM),
        ],
        out_specs=pl.BlockSpec(memory_space=pl.ANY),
        scratch_shapes=[
            pltpu.VMEM((BLK, N), jnp.float32),
            pltpu.SemaphoreType.DMA((N_SEMS,)),
            pltpu.SemaphoreType.DMA((N_SEMS,)),
            pltpu.SemaphoreType.DMA,
        ],
        compiler_params=pltpu.CompilerParams(collective_id=0),
    )(own, send)


def kernel(dy, W):
    x = lax.axis_index("x")
    y = lax.axis_index("y")
    z = lax.axis_index("z")
    m = 2 * y + z

    lo = lax.dynamic_slice_in_dim(dy, m * BLK, BLK, axis=0)
    hi = lax.dynamic_slice_in_dim(dy, (m + 4) * BLK, BLK, axis=0)
    p_lo = lo @ W.T
    p_hi = hi @ W.T

    own = jnp.where(x == 0, p_lo, p_hi)
    send = jnp.where(x == 0, p_hi, p_lo)
    return _allreduce(own, send)


# device time: 510192 ns/iter; 1.2147x vs baseline; 1.2147x over previous
import jax
import jax.numpy as jnp
from jax import lax
from jax.experimental import pallas as pl
from jax.experimental.pallas import tpu as pltpu

M = 4096
N = 4096
K = 8192
BLK = M // 8

AXIS_W = {"x": 4, "y": 2, "z": 1}

STRIPES = (
    (0,    1024, ("x", "z", "y")),
    (1024, 1024, ("x", "z", "y")),
    (2048,  512, ("y", "x", "z")),
    (2560,  512, ("y", "x", "z")),
    (3072,  512, ("z", "y", "x")),
    (3584,  512, ("z", "y", "x")),
)
N_STRIPES = len(STRIPES)
WMAX = 1024
KT = 2048
NKT = K // KT
TOT = N_STRIPES * NKT

def _stage_slot(s, j):
    return (4 * s + 6, 4 * s + 10, 4 * s + 16)[j]

N_AG = 30


def _runs(exchanged):
    offs = [0]
    for a in exchanged:
        w = AXIS_W[a]
        offs = sorted(o + d for o in offs for d in (0, w))
    runs = []
    start, length = offs[0], 1
    for o in offs[1:]:
        if o == start + length:
            length += 1
        else:
            runs.append((start, length))
            start, length = o, 1
    runs.append((start, length))
    return runs


def _fused_body(dy_ref, w_ref, out_ref, a_buf, b_buf, acc, rs_buf,
                a_sems, b_sems, rs_send, rs_recv, ag_send, ag_recv, cp_sem):
    x = lax.axis_index("x")
    y = lax.axis_index("y")
    z = lax.axis_index("z")
    idx = {"x": x, "y": y, "z": z}
    m = 2 * y + z
    b_own = 4 * x + m
    own_rows = pl.ds(b_own * BLK, BLK)

    def nbr_of(a):
        return tuple(1 - idx[ax] if ax == a else idx[ax] for ax in "xyz")

    def start_loads(g):
        s, kt = divmod(g, NKT)
        c0, w, _ = STRIPES[s]
        pa = g % 2
        descs = []
        for i, row0 in enumerate((m * BLK, (m + 4) * BLK)):
            d = pltpu.make_async_copy(
                dy_ref.at[pl.ds(row0, BLK), pl.ds(kt * KT, KT)],
                a_buf.at[pa, pl.ds(i * BLK, BLK), :],
                a_sems.at[pa, i],
            )
            d.start()
            descs.append(d)
        d = pltpu.make_async_copy(
            w_ref.at[pl.ds(c0, w), pl.ds(kt * KT, KT)],
            b_buf.at[pa, pl.ds(0, w), :],
            b_sems.at[pa],
        )
        d.start()
        descs.append(d)
        return descs

    ag_i = [0]
    exchanged = {}
    hop_descs = {}
    rs_descs = {}

    def start_hop(s):
        c0, w, order = STRIPES[s]
        ex = exchanged[s]
        a = order[len(ex)]
        base = sum(AXIS_W[ax] * idx[ax] for ax in "xyz" if ax not in ex)
        cols = pl.ds(c0, w)
        descs = []
        for off, length in _runs(ex):
            rows = pl.ds((base + off) * BLK, length * BLK)
            d = pltpu.make_async_remote_copy(
                src_ref=out_ref.at[rows, cols],
                dst_ref=out_ref.at[rows, cols],
                send_sem=ag_send.at[ag_i[0]],
                recv_sem=ag_recv.at[ag_i[0]],
                device_id=nbr_of(a),
                device_id_type=pl.DeviceIdType.MESH,
            )
            d.start()
            descs.append(d)
            ag_i[0] += 1
        exchanged[s] = ex + (a,)
        hop_descs[s] = descs

    def stage(s, j):
        c0, w, _ = STRIPES[s]
        if j == 0:
            rs_descs[s].wait()
            sa = s % 2
            rs_buf[s, :, pl.ds(0, w)] = (
                rs_buf[s, :, pl.ds(0, w)]
                + acc[sa, pl.ds(x * BLK, BLK), pl.ds(0, w)]
            )
            cp = pltpu.make_async_copy(
                rs_buf.at[s, :, pl.ds(0, w)],
                out_ref.at[own_rows, pl.ds(c0, w)],
                cp_sem,
            )
            cp.start()
            cp.wait()
            exchanged[s] = ()
            start_hop(s)
        else:
            for d in hop_descs[s]:
                d.wait()
            start_hop(s)

    slot_stages = {}
    epilogue = []
    for s in range(N_STRIPES):
        for j in range(3):
            g = _stage_slot(s, j)
            if g < TOT:
                slot_stages.setdefault(g, []).append((s, j))
            else:
                epilogue.append((g, s, j))
    epilogue.sort()

    barrier = pltpu.get_barrier_semaphore()
    for a in "xyz":
        pl.semaphore_signal(barrier, inc=1, device_id=nbr_of(a),
                            device_id_type=pl.DeviceIdType.MESH)
    pl.semaphore_wait(barrier, 3)

    pending = start_loads(0)
    for g in range(TOT):
        s, kt = divmod(g, NKT)
        c0, w, _ = STRIPES[s]
        for d in pending:
            d.wait()
        nxt = start_loads(g + 1) if g + 1 < TOT else ()
        pa = g % 2
        prod = lax.dot_general(
            a_buf[pa],
            b_buf[pa, pl.ds(0, w), :],
            (((1,), (1,)), ((), ())),
            preferred_element_type=jnp.float32,
        )
        sa = s % 2
        if kt == 0:
            acc[sa, :, pl.ds(0, w)] = prod
        else:
            acc[sa, :, pl.ds(0, w)] = acc[sa, :, pl.ds(0, w)] + prod
        if kt == NKT - 1:
            d = pltpu.make_async_remote_copy(
                src_ref=acc.at[sa, pl.ds((1 - x) * BLK, BLK), pl.ds(0, w)],
                dst_ref=rs_buf.at[s, :, pl.ds(0, w)],
                send_sem=rs_send.at[s],
                recv_sem=rs_recv.at[s],
                device_id=nbr_of("x"),
                device_id_type=pl.DeviceIdType.MESH,
            )
            d.start()
            rs_descs[s] = d
        for sj in slot_stages.get(g, ()):
            stage(*sj)
        pending = nxt

    for _, s, j in epilogue:
        stage(s, j)
    for s in range(N_STRIPES):
        for d in hop_descs[s]:
            d.wait()


def kernel(dy, W):
    return pl.pallas_call(
        _fused_body,
        out_shape=jax.ShapeDtypeStruct((M, N), jnp.float32),
        in_specs=[
            pl.BlockSpec(memory_space=pl.ANY),
            pl.BlockSpec(memory_space=pl.ANY),
        ],
        out_specs=pl.BlockSpec(memory_space=pl.ANY),
        scratch_shapes=[
            pltpu.VMEM((2, 2 * BLK, KT), jnp.float32),
            pltpu.VMEM((2, WMAX, KT), jnp.float32),
            pltpu.VMEM((2, 2 * BLK, WMAX), jnp.float32),
            pltpu.VMEM((N_STRIPES, BLK, WMAX), jnp.float32),
            pltpu.SemaphoreType.DMA((2, 2)),
            pltpu.SemaphoreType.DMA((2,)),
            pltpu.SemaphoreType.DMA((N_STRIPES,)),
            pltpu.SemaphoreType.DMA((N_STRIPES,)),
            pltpu.SemaphoreType.DMA((N_AG,)),
            pltpu.SemaphoreType.DMA((N_AG,)),
            pltpu.SemaphoreType.DMA,
        ],
        compiler_params=pltpu.CompilerParams(
            collective_id=0,
            vmem_limit_bytes=63 * 1024 * 1024,
        ),
    )(dy, W)
